# baseline (speedup 1.0000x reference)
import functools

import jax
import jax.numpy as jnp
import numpy as np
from jax import lax
from jax.experimental import pallas as pl
from jax.experimental.pallas import tpu as pltpu
from jax.experimental.pallas import tpu_sc as plsc

_LANES = 16
_NBUF = 2
_HI = np.int32(-65536)
_RND = np.int32(0x8000)
_S16 = np.int32(16)


@functools.lru_cache(maxsize=None)
def _build(n: int, chv: int, nw: int, unroll: int):
    che = chv * _LANES
    nch = n // che
    assert nch * che == n
    iters = (nch + nw - 1) // nw
    outer_iters = (iters + _NBUF - 1) // _NBUF

    mesh = plsc.VectorSubcoreMesh(core_axis_name="c", subcore_axis_name="s")
    nc = 2

    @functools.partial(
        pl.kernel,
        out_type=jax.ShapeDtypeStruct((n,), jnp.int32),
        mesh=mesh,
        compiler_params=pltpu.CompilerParams(needs_layout_passes=False),
        scratch_types=[
            pltpu.VMEM((64,), jnp.float32),
            pltpu.VMEM((64,), jnp.float32),
            pltpu.VMEM((64,), jnp.int32),
        ] + [pltpu.VMEM((che,), jnp.int32) for _ in range(_NBUF)]
          + [pltpu.VMEM((che,), jnp.int32) for _ in range(_NBUF)]
          + [pltpu.SemaphoreType.DMA for _ in range(2 * _NBUF)],
    )
    def k(sp_hbm, scales_hbm, shifts_hbm, sb_hbm,
          scales_v, shifts_v, tab_v, sp0, sp1, ob0, ob1,
          isem0, isem1, osem0, osem1):
        sp_bufs = [sp0, sp1]
        o_bufs = [ob0, ob1]
        in_sems = [isem0, isem1]
        out_sems = [osem0, osem1]

        w = lax.axis_index("s") * nc + lax.axis_index("c")
        pltpu.sync_copy(scales_hbm, scales_v)
        pltpu.sync_copy(shifts_hbm, shifts_v)

        for j in range(64 // _LANES):
            sv = plsc.bitcast(scales_v[pl.ds(j * _LANES, _LANES)], jnp.int32)
            bv = plsc.bitcast(shifts_v[pl.ds(j * _LANES, _LANES)], jnp.int32)
            hi = (sv + _RND) & _HI
            lo = lax.shift_right_logical(bv + _RND, _S16)
            tab_v[pl.ds(j * _LANES, _LANES)] = hi | lo

        def start_in(k_, slot):
            ci = w + k_ * nw

            @pl.when(ci < nch)
            def _():
                base = ci * che
                pltpu.async_copy(sp_hbm.at[pl.ds(base, che)],
                                 sp_bufs[slot], in_sems[slot])

        def step(k_, slot):
            ci = w + k_ * nw

            @pl.when(ci < nch)
            def _():
                base = ci * che
                pltpu.make_async_copy(sp_hbm.at[pl.ds(base, che)],
                                      sp_bufs[slot], in_sems[slot]).wait()
                @pl.when(k_ >= _NBUF)
                def _():
                    pltpu.make_async_copy(o_bufs[slot],
                                          sb_hbm.at[pl.ds(base, che)],
                                          out_sems[slot]).wait()

                sp_b, o_b = sp_bufs[slot], o_bufs[slot]

                @plsc.parallel_loop(0, chv, unroll=unroll)
                def _(i):
                    off = i * _LANES
                    idx = sp_b[pl.ds(off, _LANES)]
                    o_b[pl.ds(off, _LANES)] = plsc.load_gather(tab_v, [idx])

                pltpu.async_copy(o_b, sb_hbm.at[pl.ds(base, che)],
                                 out_sems[slot])
                start_in(k_ + _NBUF, slot)

        for s in range(_NBUF):
            start_in(s, s)

        def outer(kk, carry):
            for s in range(_NBUF):
                step(kk * _NBUF + s, s)
            return carry

        lax.fori_loop(0, outer_iters, outer, 0)

        my_iters = (nch - w + nw - 1) // nw

        for s in range(_NBUF):
            for d in range(1, _NBUF + 1):
                k_ = my_iters - d

                @pl.when((k_ >= 0) & (k_ % _NBUF == s))
                def _():
                    pltpu.make_async_copy(o_bufs[s],
                                          sb_hbm.at[pl.ds(0, che)],
                                          out_sems[s]).wait()

    return k


def kernel(x, species, scales, shifts):
    n = x.shape[0]
    k = _build(n, 500, 32, 8)
    sb = k(species, scales, shifts)
    u = sb.reshape(n, 1)
    s = lax.bitcast_convert_type(u & _HI, jnp.float32)
    b = lax.bitcast_convert_type(lax.shift_left(u, _S16), jnp.float32)
    return s * x + b


# speedup vs baseline: 1.0856x; 1.0856x over previous
import functools

import jax
import jax.numpy as jnp
from jax import lax
from jax.experimental import pallas as pl
from jax.experimental.pallas import tpu as pltpu
from jax.experimental.pallas import tpu_sc as plsc

_LANES = 16
_NBUF = 2


@functools.lru_cache(maxsize=None)
def _build(n: int, chv: int, nw: int, unroll: int):
    che = chv * _LANES
    nch = n // che
    assert nch * che == n
    iters = (nch + nw - 1) // nw
    outer_iters = (iters + _NBUF - 1) // _NBUF

    mesh = plsc.VectorSubcoreMesh(core_axis_name="c", subcore_axis_name="s")
    nc = 2

    @functools.partial(
        pl.kernel,
        out_type=(jax.ShapeDtypeStruct((n,), jnp.float32),
                  jax.ShapeDtypeStruct((n,), jnp.float32)),
        mesh=mesh,
        compiler_params=pltpu.CompilerParams(needs_layout_passes=False),
        scratch_types=[
            pltpu.VMEM((64,), jnp.float32),
            pltpu.VMEM((64,), jnp.float32),
        ] + [pltpu.VMEM((che,), jnp.int32) for _ in range(_NBUF)]
          + [pltpu.VMEM((che,), jnp.float32) for _ in range(_NBUF)]
          + [pltpu.VMEM((che,), jnp.float32) for _ in range(_NBUF)]
          + [pltpu.SemaphoreType.DMA for _ in range(2 * _NBUF)],
    )
    def k(sp_hbm, scales_hbm, shifts_hbm, s_hbm, b_hbm,
          scales_v, shifts_v, sp0, sp1, sb0, sb1, bb0, bb1,
          isem0, isem1, osem0, osem1):
        sp_bufs = [sp0, sp1]
        s_bufs = [sb0, sb1]
        b_bufs = [bb0, bb1]
        in_sems = [isem0, isem1]
        out_sems = [osem0, osem1]

        w = lax.axis_index("s") * nc + lax.axis_index("c")
        pltpu.sync_copy(scales_hbm, scales_v)
        pltpu.sync_copy(shifts_hbm, shifts_v)

        def start_in(k_, slot):
            ci = w + k_ * nw

            @pl.when(ci < nch)
            def _():
                base = ci * che
                pltpu.async_copy(sp_hbm.at[pl.ds(base, che)],
                                 sp_bufs[slot], in_sems[slot])

        def step(k_, slot):
            ci = w + k_ * nw

            @pl.when(ci < nch)
            def _():
                base = ci * che
                pltpu.make_async_copy(sp_hbm.at[pl.ds(base, che)],
                                      sp_bufs[slot], in_sems[slot]).wait()
                @pl.when(k_ >= _NBUF)
                def _():
                    pltpu.make_async_copy(s_bufs[slot],
                                          s_hbm.at[pl.ds(base, che)],
                                          out_sems[slot]).wait()
                    pltpu.make_async_copy(b_bufs[slot],
                                          b_hbm.at[pl.ds(base, che)],
                                          out_sems[slot]).wait()

                sp_b, s_b, b_b = sp_bufs[slot], s_bufs[slot], b_bufs[slot]

                @plsc.parallel_loop(0, chv, unroll=unroll)
                def _(i):
                    off = i * _LANES
                    idx = sp_b[pl.ds(off, _LANES)]
                    s_b[pl.ds(off, _LANES)] = plsc.load_gather(scales_v, [idx])
                    b_b[pl.ds(off, _LANES)] = plsc.load_gather(shifts_v, [idx])

                pltpu.async_copy(s_b, s_hbm.at[pl.ds(base, che)],
                                 out_sems[slot])
                pltpu.async_copy(b_b, b_hbm.at[pl.ds(base, che)],
                                 out_sems[slot])
                start_in(k_ + _NBUF, slot)

        for s in range(_NBUF):
            start_in(s, s)

        def outer(kk, carry):
            for s in range(_NBUF):
                step(kk * _NBUF + s, s)
            return carry

        lax.fori_loop(0, outer_iters, outer, 0)

        my_iters = (nch - w + nw - 1) // nw

        for s in range(_NBUF):
            for d in range(1, _NBUF + 1):
                k_ = my_iters - d

                @pl.when((k_ >= 0) & (k_ % _NBUF == s))
                def _():
                    pltpu.make_async_copy(s_bufs[s],
                                          s_hbm.at[pl.ds(0, che)],
                                          out_sems[s]).wait()
                    pltpu.make_async_copy(b_bufs[s],
                                          b_hbm.at[pl.ds(0, che)],
                                          out_sems[s]).wait()

    return k


def kernel(x, species, scales, shifts):
    n = x.shape[0]
    h = n // 2
    k = _build(n, 500, 32, 8)
    s_arr, b_arr = k(species, scales, shifts)
    o_lo = s_arr[:h].reshape(h, 1) * x[:h] + b_arr[:h].reshape(h, 1)
    o_hi = s_arr[h:].reshape(h, 1) * x[h:] + b_arr[h:].reshape(h, 1)
    return jnp.concatenate([o_lo, o_hi], axis=0)


# speedup vs baseline: 1.0956x; 1.0091x over previous
import functools

import jax
import jax.numpy as jnp
from jax import lax
from jax.experimental import pallas as pl
from jax.experimental.pallas import tpu as pltpu
from jax.experimental.pallas import tpu_sc as plsc

_LANES = 16
_NBUF = 2


@functools.lru_cache(maxsize=None)
def _build(n: int, chv: int, nw: int, unroll: int, c0: int, c1: int):
    che = chv * _LANES
    assert (n // che) * che == n
    nch = c1 - c0
    nout = nch * che
    iters = (nch + nw - 1) // nw
    outer_iters = (iters + _NBUF - 1) // _NBUF

    mesh = plsc.VectorSubcoreMesh(core_axis_name="c", subcore_axis_name="s")
    nc = 2

    @functools.partial(
        pl.kernel,
        out_type=(jax.ShapeDtypeStruct((nout,), jnp.float32),
                  jax.ShapeDtypeStruct((nout,), jnp.float32)),
        mesh=mesh,
        compiler_params=pltpu.CompilerParams(needs_layout_passes=False),
        scratch_types=[
            pltpu.VMEM((64,), jnp.float32),
            pltpu.VMEM((64,), jnp.float32),
        ] + [pltpu.VMEM((che,), jnp.int32) for _ in range(_NBUF)]
          + [pltpu.VMEM((che,), jnp.float32) for _ in range(_NBUF)]
          + [pltpu.VMEM((che,), jnp.float32) for _ in range(_NBUF)]
          + [pltpu.SemaphoreType.DMA for _ in range(2 * _NBUF)],
    )
    def k(sp_hbm, scales_hbm, shifts_hbm, s_hbm, b_hbm,
          scales_v, shifts_v, sp0, sp1, sb0, sb1, bb0, bb1,
          isem0, isem1, osem0, osem1):
        sp_bufs = [sp0, sp1]
        s_bufs = [sb0, sb1]
        b_bufs = [bb0, bb1]
        in_sems = [isem0, isem1]
        out_sems = [osem0, osem1]

        w = lax.axis_index("s") * nc + lax.axis_index("c")
        pltpu.sync_copy(scales_hbm, scales_v)
        pltpu.sync_copy(shifts_hbm, shifts_v)

        def start_in(k_, slot):
            ci = w + k_ * nw

            @pl.when(ci < nch)
            def _():
                pltpu.async_copy(sp_hbm.at[pl.ds((c0 + ci) * che, che)],
                                 sp_bufs[slot], in_sems[slot])

        def step(k_, slot):
            ci = w + k_ * nw

            @pl.when(ci < nch)
            def _():
                base = ci * che
                pltpu.make_async_copy(sp_hbm.at[pl.ds(base, che)],
                                      sp_bufs[slot], in_sems[slot]).wait()
                @pl.when(k_ >= _NBUF)
                def _():
                    pltpu.make_async_copy(s_bufs[slot],
                                          s_hbm.at[pl.ds(base, che)],
                                          out_sems[slot]).wait()
                    pltpu.make_async_copy(b_bufs[slot],
                                          b_hbm.at[pl.ds(base, che)],
                                          out_sems[slot]).wait()

                sp_b, s_b, b_b = sp_bufs[slot], s_bufs[slot], b_bufs[slot]

                @plsc.parallel_loop(0, chv, unroll=unroll)
                def _(i):
                    off = i * _LANES
                    idx = sp_b[pl.ds(off, _LANES)]
                    s_b[pl.ds(off, _LANES)] = plsc.load_gather(scales_v, [idx])
                    b_b[pl.ds(off, _LANES)] = plsc.load_gather(shifts_v, [idx])

                pltpu.async_copy(s_b, s_hbm.at[pl.ds(base, che)],
                                 out_sems[slot])
                pltpu.async_copy(b_b, b_hbm.at[pl.ds(base, che)],
                                 out_sems[slot])
                start_in(k_ + _NBUF, slot)

        for s in range(_NBUF):
            start_in(s, s)

        def outer(kk, carry):
            for s in range(_NBUF):
                step(kk * _NBUF + s, s)
            return carry

        lax.fori_loop(0, outer_iters, outer, 0)

        my_iters = (nch - w + nw - 1) // nw

        for s in range(_NBUF):
            for d in range(1, _NBUF + 1):
                k_ = my_iters - d

                @pl.when((k_ >= 0) & (k_ % _NBUF == s))
                def _():
                    pltpu.make_async_copy(s_bufs[s],
                                          s_hbm.at[pl.ds(0, che)],
                                          out_sems[s]).wait()
                    pltpu.make_async_copy(b_bufs[s],
                                          b_hbm.at[pl.ds(0, che)],
                                          out_sems[s]).wait()

    return k


def kernel(x, species, scales, shifts):
    n = x.shape[0]
    chv = 1000
    nch = n // (chv * _LANES)
    k = _build(n, chv, 32, 8, 0, nch)
    s_arr, b_arr = k(species, scales, shifts)
    return s_arr.reshape(n, 1) * x + b_arr.reshape(n, 1)
